# baseline (speedup 1.0000x reference)
import jax
import jax.numpy as jnp
from jax import lax
from jax.experimental import pallas as pl
from jax.experimental.pallas import tpu as pltpu
from jax.experimental.pallas import tpu_sc as plsc

_N = 10000
_E = 320000
_D = 128
_H = 128
_C = 64
_G = 16
_NPAD = 10240
_EP = 327680
_EPT = 10240
_NSLAB = 5
_ES0 = 5
_ES1 = 10 - _ES0
_SLAB = 2048
_CHUNK = 64
_NCH = _SLAB // _CHUNK
_BLK = 1000

_mesh = plsc.VectorSubcoreMesh(core_axis_name="c", subcore_axis_name="s")


def _edge_body(src_hbm, dst_hbm, attr_hbm, par_hbm,
               ew_hbm, degp_hbm,
               src_v, dst_v, attr_v, ew_v, par_v, deg_local, tmp_v, acc_v,
               deg_sh):
    c = lax.axis_index("c")
    s = lax.axis_index("s")
    base_slab = jnp.where(c == 0, s * _ES0, 16 * _ES0 + s * _ES1)
    nsl = jnp.where(c == 0, _ES0, _ES1)
    zero16 = jnp.zeros((16,), jnp.float32)

    def zbody(i, _):
        deg_local[pl.ds(i * 16, 16)] = zero16
        return 0
    lax.fori_loop(0, _NPAD // 16, zbody, 0)

    pltpu.sync_copy(par_hbm, par_v)
    pv = par_v[...]
    w10 = pv[0]
    w11 = pv[1]
    w12 = pv[2]
    b1 = pv[3]
    w20 = pv[4]
    b2 = pv[5]

    def sbbody(sb, _):
        sbase = (base_slab + sb) * _SLAB
        pltpu.sync_copy(src_hbm.at[pl.ds(sbase, _SLAB)], src_v)
        pltpu.sync_copy(dst_hbm.at[pl.ds(sbase, _SLAB)], dst_v)
        pltpu.sync_copy(attr_hbm.at[pl.ds(sbase, _SLAB)], attr_v)

        def gbody(g, _):
            col = g * 16
            di = dst_v[pl.ds(col, 16)]
            sf = src_v[pl.ds(col, 16)].astype(jnp.float32)
            df = di.astype(jnp.float32)
            af = attr_v[pl.ds(col, 16)]
            t = jnp.maximum(sf * w10 + df * w11 + af * w12 + b1, 0.0)
            ew = 1.0 / (1.0 + jnp.exp(-(t * w20 + b2)))
            ew_v[pl.ds(col, 16)] = ew
            plsc.addupdate_scatter(deg_local, [di], ew)
            return 0
        lax.fori_loop(0, _SLAB // 16, gbody, 0)
        pltpu.sync_copy(ew_v, ew_hbm.at[pl.ds(sbase, _SLAB)])
        return 0
    lax.fori_loop(0, nsl, sbbody, 0)

    pltpu.sync_copy(deg_local, deg_sh.at[s, 0])
    plsc.subcore_barrier()
    sl = s * 640

    def z2(i, _):
        acc_v[pl.ds(i * 16, 16)] = zero16
        return 0
    lax.fori_loop(0, 40, z2, 0)
    for t in range(16):
        pltpu.sync_copy(deg_sh.at[t, 0, pl.ds(sl, 640)], tmp_v)

        def abody(i, _):
            acc_v[pl.ds(i * 16, 16)] = (acc_v[pl.ds(i * 16, 16)]
                                        + tmp_v[pl.ds(i * 16, 16)])
            return 0
        lax.fori_loop(0, 40, abody, 0)
    pltpu.sync_copy(acc_v, degp_hbm.at[c, 0, pl.ds(sl, 640)])


_edge_kernel = pl.kernel(
    _edge_body,
    compiler_params=pltpu.CompilerParams(needs_layout_passes=False),
    out_type=[jax.ShapeDtypeStruct((_EP,), jnp.float32),
              jax.ShapeDtypeStruct((2, 1, _NPAD), jnp.float32)],
    mesh=_mesh,
    scratch_types=[
        pltpu.VMEM((_SLAB,), jnp.int32),
        pltpu.VMEM((_SLAB,), jnp.int32),
        pltpu.VMEM((_SLAB,), jnp.float32),
        pltpu.VMEM((_SLAB,), jnp.float32),
        pltpu.VMEM((16,), jnp.float32),
        pltpu.VMEM((_NPAD,), jnp.float32),
        pltpu.VMEM((640,), jnp.float32),
        pltpu.VMEM((640,), jnp.float32),
        pltpu.VMEM_SHARED((16, 1, _NPAD), jnp.float32),
    ],
)


def _msg_body(src_hbm, dst3_hbm, ew_hbm, gs_hbm, zeros_hbm,
              msgp_hbm,
              src_v, dst_v, ew_v, rows_0, rows_1, rows_2, rows_3,
              sem_g, sem_s, msg_sh):
    c = lax.axis_index("c")
    s = lax.axis_index("s")
    nb = s * 640
    rows = (rows_0, rows_1, rows_2, rows_3)

    def gissue(ci, buf):
        idx = src_v.at[pl.ds(ci * _CHUNK, _CHUNK)]
        pltpu.async_copy(gs_hbm.at[idx], buf, sem_g)

    def gwait(buf):
        pltpu.make_async_copy(gs_hbm.at[pl.ds(0, _CHUNK)], buf, sem_g).wait()

    def sissue(ci, buf):
        pltpu.async_copy(buf, msg_sh.at[dst_v.at[ci]], sem_s, add=True)

    def swait(ci, buf):
        pltpu.make_async_copy(buf, msg_sh.at[dst_v.at[ci]], sem_s).wait()

    @pl.when(c == 0)
    def _():
        pltpu.sync_copy(zeros_hbm.at[pl.ds(nb, 640)],
                        msg_sh.at[pl.ds(nb, 640)])
        plsc.subcore_barrier()

        def sbbody(sb, _):
            base_slab = s * 10 + sb
            sbase = base_slab * _SLAB
            pltpu.sync_copy(src_hbm.at[pl.ds(sbase, _SLAB)], src_v)
            pltpu.sync_copy(dst3_hbm.at[base_slab], dst_v)
            pltpu.sync_copy(ew_hbm.at[pl.ds(sbase, _SLAB)], ew_v)
            gissue(0, rows_0)

            def quadbody(cq, _):
                for q in range(4):
                    ci = 4 * cq + q
                    buf = rows[q]
                    nxt = rows[(q + 1) % 4]

                    @pl.when(ci >= 3)
                    def _():
                        swait(ci - 3, nxt)

                    @pl.when(ci < _NCH - 1)
                    def _():
                        gissue(ci + 1, nxt)
                    gwait(buf)

                    def ebody(b, _):
                        e16 = jnp.full((16,), ci * _CHUNK + b, jnp.int32)
                        w = plsc.load_gather(ew_v, [e16])
                        for j in range(8):
                            buf[b, pl.ds(j * 16, 16)] = (
                                buf[b, pl.ds(j * 16, 16)] * w)
                        return 0
                    lax.fori_loop(0, _CHUNK, ebody, 0, unroll=2)
                    sissue(ci, buf)
                return 0
            lax.fori_loop(0, _NCH // 4, quadbody, 0)
            swait(_NCH - 3, rows[1])
            swait(_NCH - 2, rows[2])
            swait(_NCH - 1, rows[3])
            return 0
        lax.fori_loop(0, 10, sbbody, 0)

        plsc.subcore_barrier()
        pltpu.sync_copy(msg_sh.at[pl.ds(nb, 640)],
                        msgp_hbm.at[pl.ds(nb, 640)])


_msg_kernel = pl.kernel(
    _msg_body,
    compiler_params=pltpu.CompilerParams(needs_layout_passes=False),
    out_type=jax.ShapeDtypeStruct((_NPAD, _H), jnp.float32),
    mesh=_mesh,
    scratch_types=[
        pltpu.VMEM((_SLAB,), jnp.int32),
        pltpu.VMEM((_NCH, _CHUNK), jnp.int32),
        pltpu.VMEM((_SLAB,), jnp.float32),
        pltpu.VMEM((_CHUNK, _H), jnp.float32),
        pltpu.VMEM((_CHUNK, _H), jnp.float32),
        pltpu.VMEM((_CHUNK, _H), jnp.float32),
        pltpu.VMEM((_CHUNK, _H), jnp.float32),
        pltpu.SemaphoreType.DMA,
        pltpu.SemaphoreType.DMA,
        pltpu.VMEM_SHARED((_NPAD, _H), jnp.float32),
    ],
)


def _k2_body(dp0, dp1, x, w, dis_out, gs_out):
    deg = dp0[...] + dp1[...] + 1.0
    dis = lax.rsqrt(jnp.maximum(deg, 1e-12))
    dis_out[...] = dis
    gs_out[...] = dis * jnp.dot(x[...], w[...],
                                preferred_element_type=jnp.float32)


_k2 = pl.pallas_call(
    _k2_body,
    grid=(_N // _BLK,),
    in_specs=[
        pl.BlockSpec((_BLK, 1), lambda i: (i, 0)),
        pl.BlockSpec((_BLK, 1), lambda i: (i, 0)),
        pl.BlockSpec((_BLK, _D), lambda i: (i, 0)),
        pl.BlockSpec((_D, _H), lambda i: (0, 0)),
    ],
    out_specs=[pl.BlockSpec((_BLK, 1), lambda i: (i, 0)),
               pl.BlockSpec((_BLK, _H), lambda i: (i, 0))],
    out_shape=[jax.ShapeDtypeStruct((_N, 1), jnp.float32),
               jax.ShapeDtypeStruct((_N, _H), jnp.float32)],
)


def _k4_body(dis, m0, gs, b, w, gs2_out):
    h = jnp.maximum(dis[...] * (m0[...] + gs[...]) + b[...], 0.0)
    gs2_out[...] = dis[...] * jnp.dot(h, w[...],
                                      preferred_element_type=jnp.float32)


_k4 = pl.pallas_call(
    _k4_body,
    grid=(_N // _BLK,),
    in_specs=[
        pl.BlockSpec((_BLK, 1), lambda i: (i, 0)),
        pl.BlockSpec((_BLK, _H), lambda i: (i, 0)),
        pl.BlockSpec((_BLK, _H), lambda i: (i, 0)),
        pl.BlockSpec((1, _H), lambda i: (0, 0)),
        pl.BlockSpec((_H, _H), lambda i: (0, 0)),
    ],
    out_specs=pl.BlockSpec((_BLK, _H), lambda i: (i, 0)),
    out_shape=jax.ShapeDtypeStruct((_N, _H), jnp.float32),
)


def _k6_body(dis, m0, gs, b, batch, lw, lb, out, sums, cnts):
    i = pl.program_id(0)

    @pl.when(i == 0)
    def _():
        sums[...] = jnp.zeros_like(sums)
        cnts[...] = jnp.zeros_like(cnts)

    h = jnp.maximum(dis[...] * (m0[...] + gs[...]) + b[...], 0.0)
    gi = lax.broadcasted_iota(jnp.int32, (_BLK, _G), 1)
    oh = (gi == batch[...]).astype(jnp.float32)
    dn = (((0,), (0,)), ((), ()))
    sums[...] += lax.dot_general(oh, h, dn,
                                 preferred_element_type=jnp.float32)
    cnts[...] += lax.dot_general(oh, jnp.ones((_BLK, 1), jnp.float32), dn,
                                 preferred_element_type=jnp.float32)

    @pl.when(i == pl.num_programs(0) - 1)
    def _():
        pooled = sums[...] / jnp.maximum(cnts[...], 1.0)
        out[...] = jnp.dot(pooled, lw[...],
                           preferred_element_type=jnp.float32) + lb[...]


_k6 = pl.pallas_call(
    _k6_body,
    grid=(_N // _BLK,),
    in_specs=[
        pl.BlockSpec((_BLK, 1), lambda i: (i, 0)),
        pl.BlockSpec((_BLK, _H), lambda i: (i, 0)),
        pl.BlockSpec((_BLK, _H), lambda i: (i, 0)),
        pl.BlockSpec((1, _H), lambda i: (0, 0)),
        pl.BlockSpec((_BLK, 1), lambda i: (i, 0)),
        pl.BlockSpec((_H, _C), lambda i: (0, 0)),
        pl.BlockSpec((1, _C), lambda i: (0, 0)),
    ],
    out_specs=pl.BlockSpec((_G, _C), lambda i: (0, 0)),
    out_shape=jax.ShapeDtypeStruct((_G, _C), jnp.float32),
    scratch_shapes=[pltpu.VMEM((_G, _H), jnp.float32),
                    pltpu.VMEM((_G, 1), jnp.float32)],
)


def kernel(x, edge_index, edge_attr, batch, ee_w1, ee_b1, ee_w2, ee_b2,
           conv1_w, conv1_b, stem_w, stem_b, lin_w, lin_b):
    npad = _EP - _E
    srcp = jnp.concatenate([edge_index[0], jnp.zeros((npad,), jnp.int32)])
    dstp = jnp.concatenate([edge_index[1],
                            jnp.full((npad,), _N + 200, jnp.int32)])
    attrp = jnp.concatenate([edge_attr[:, 0],
                             jnp.zeros((npad,), jnp.float32)])
    dst3d = dstp.reshape(32 * _NSLAB, _NCH, _CHUNK)
    par = jnp.concatenate([ee_w1[:, 0], ee_b1, ee_w2[0], ee_b2,
                           jnp.zeros((10,), jnp.float32)])

    ew1d, degp = _edge_kernel(srcp, dstp, attrp, par)
    dp0 = degp[0, 0, :_N, None]
    dp1 = degp[1, 0, :_N, None]
    dis, gs1 = _k2(dp0, dp1, x, conv1_w)

    zeros = jnp.zeros((_NPAD, _H), jnp.float32)
    msgp1 = _msg_kernel(srcp, dst3d, ew1d, gs1, zeros)
    gs2 = _k4(dis, msgp1[:_N], gs1, conv1_b.reshape(1, _H), stem_w)
    msgp2 = _msg_kernel(srcp, dst3d, ew1d, gs2, zeros)
    out = _k6(dis, msgp2[:_N], gs2, stem_b.reshape(1, _H),
              batch.reshape(_N, 1), lin_w, lin_b.reshape(1, _C))
    return out


# speedup vs baseline: 1.3535x; 1.3535x over previous
import jax
import jax.numpy as jnp
from jax import lax
from jax.experimental import pallas as pl
from jax.experimental.pallas import tpu as pltpu
from jax.experimental.pallas import tpu_sc as plsc

_N = 10000
_E = 320000
_D = 128
_H = 128
_C = 64
_G = 16
_NPAD = 10240
_EP = 327680
_EPT = 10240
_NSLAB = 5
_S0 = 9
_S1 = 10 - _S0
_ES0 = 5
_ES1 = 10 - _ES0
_SLAB = 2048
_CHUNK = 64
_NCH = _SLAB // _CHUNK
_BLK = 1000

_mesh = plsc.VectorSubcoreMesh(core_axis_name="c", subcore_axis_name="s")


def _edge_body(src_hbm, dst_hbm, attr_hbm, par_hbm,
               ew_hbm, degp_hbm,
               src_v, dst_v, attr_v, ew_v, par_v, deg_local, tmp_v, acc_v,
               deg_sh):
    c = lax.axis_index("c")
    s = lax.axis_index("s")
    base_slab = jnp.where(c == 0, s * _ES0, 16 * _ES0 + s * _ES1)
    nsl = jnp.where(c == 0, _ES0, _ES1)
    zero16 = jnp.zeros((16,), jnp.float32)

    def zbody(i, _):
        deg_local[pl.ds(i * 16, 16)] = zero16
        return 0
    lax.fori_loop(0, _NPAD // 16, zbody, 0)

    pltpu.sync_copy(par_hbm, par_v)
    pv = par_v[...]
    w10 = pv[0]
    w11 = pv[1]
    w12 = pv[2]
    b1 = pv[3]
    w20 = pv[4]
    b2 = pv[5]

    def sbbody(sb, _):
        sbase = (base_slab + sb) * _SLAB
        pltpu.sync_copy(src_hbm.at[pl.ds(sbase, _SLAB)], src_v)
        pltpu.sync_copy(dst_hbm.at[pl.ds(sbase, _SLAB)], dst_v)
        pltpu.sync_copy(attr_hbm.at[pl.ds(sbase, _SLAB)], attr_v)

        def gbody(g, _):
            col = g * 16
            di = dst_v[pl.ds(col, 16)]
            sf = src_v[pl.ds(col, 16)].astype(jnp.float32)
            df = di.astype(jnp.float32)
            af = attr_v[pl.ds(col, 16)]
            t = jnp.maximum(sf * w10 + df * w11 + af * w12 + b1, 0.0)
            ew = 1.0 / (1.0 + jnp.exp(-(t * w20 + b2)))
            ew_v[pl.ds(col, 16)] = ew
            plsc.addupdate_scatter(deg_local, [di], ew)
            return 0
        lax.fori_loop(0, _SLAB // 16, gbody, 0)
        pltpu.sync_copy(ew_v, ew_hbm.at[pl.ds(sbase, _SLAB)])
        return 0
    lax.fori_loop(0, nsl, sbbody, 0)

    pltpu.sync_copy(deg_local, deg_sh.at[s, 0])
    plsc.subcore_barrier()
    sl = s * 640

    def z2(i, _):
        acc_v[pl.ds(i * 16, 16)] = zero16
        return 0
    lax.fori_loop(0, 40, z2, 0)
    for t in range(16):
        pltpu.sync_copy(deg_sh.at[t, 0, pl.ds(sl, 640)], tmp_v)

        def abody(i, _):
            acc_v[pl.ds(i * 16, 16)] = (acc_v[pl.ds(i * 16, 16)]
                                        + tmp_v[pl.ds(i * 16, 16)])
            return 0
        lax.fori_loop(0, 40, abody, 0)
    pltpu.sync_copy(acc_v, degp_hbm.at[c, 0, pl.ds(sl, 640)])


_edge_kernel = pl.kernel(
    _edge_body,
    compiler_params=pltpu.CompilerParams(needs_layout_passes=False),
    out_type=[jax.ShapeDtypeStruct((_EP,), jnp.float32),
              jax.ShapeDtypeStruct((2, 1, _NPAD), jnp.float32)],
    mesh=_mesh,
    scratch_types=[
        pltpu.VMEM((_SLAB,), jnp.int32),
        pltpu.VMEM((_SLAB,), jnp.int32),
        pltpu.VMEM((_SLAB,), jnp.float32),
        pltpu.VMEM((_SLAB,), jnp.float32),
        pltpu.VMEM((16,), jnp.float32),
        pltpu.VMEM((_NPAD,), jnp.float32),
        pltpu.VMEM((640,), jnp.float32),
        pltpu.VMEM((640,), jnp.float32),
        pltpu.VMEM_SHARED((16, 1, _NPAD), jnp.float32),
    ],
)


def _msg_body(src_hbm, dst3_hbm, ew_hbm, gs_hbm, zeros_hbm,
              msgp_hbm,
              src_v, dst_v, ew_v, rows_0, rows_1, rows_2, rows_3,
              sem_g, sem_s, msg_sh):
    c = lax.axis_index("c")
    s = lax.axis_index("s")
    base_slab = jnp.where(c == 0, s * _S0, 16 * _S0 + s * _S1)
    nsl = jnp.where(c == 0, _S0, _S1)
    nb = s * 640
    rows = (rows_0, rows_1, rows_2, rows_3)
    pltpu.sync_copy(zeros_hbm.at[pl.ds(nb, 640)], msg_sh.at[pl.ds(nb, 640)])
    plsc.subcore_barrier()

    def gissue(ci, buf):
        idx = src_v.at[pl.ds(ci * _CHUNK, _CHUNK)]
        pltpu.async_copy(gs_hbm.at[idx], buf, sem_g)

    def gwait(buf):
        pltpu.make_async_copy(gs_hbm.at[pl.ds(0, _CHUNK)], buf, sem_g).wait()

    def sissue(ci, buf):
        pltpu.async_copy(buf, msg_sh.at[dst_v.at[ci]], sem_s, add=True)

    def swait(ci, buf):
        pltpu.make_async_copy(buf, msg_sh.at[dst_v.at[ci]], sem_s).wait()

    def sbbody(sb, _):
        sbase = (base_slab + sb) * _SLAB
        pltpu.sync_copy(src_hbm.at[pl.ds(sbase, _SLAB)], src_v)
        pltpu.sync_copy(dst3_hbm.at[base_slab + sb], dst_v)
        pltpu.sync_copy(ew_hbm.at[pl.ds(sbase, _SLAB)], ew_v)
        gissue(0, rows_0)

        def quadbody(cq, _):
            for q in range(4):
                ci = 4 * cq + q
                buf = rows[q]
                nxt = rows[(q + 1) % 4]

                @pl.when(ci >= 3)
                def _():
                    swait(ci - 3, nxt)

                @pl.when(ci < _NCH - 1)
                def _():
                    gissue(ci + 1, nxt)
                gwait(buf)

                def ebody(b, _):
                    e16 = jnp.full((16,), ci * _CHUNK + b, jnp.int32)
                    w = plsc.load_gather(ew_v, [e16])
                    for j in range(8):
                        buf[b, pl.ds(j * 16, 16)] = (
                            buf[b, pl.ds(j * 16, 16)] * w)
                    return 0
                lax.fori_loop(0, _CHUNK, ebody, 0, unroll=2)
                sissue(ci, buf)
            return 0
        lax.fori_loop(0, _NCH // 4, quadbody, 0)
        swait(_NCH - 3, rows[1])
        swait(_NCH - 2, rows[2])
        swait(_NCH - 1, rows[3])
        return 0
    lax.fori_loop(0, nsl, sbbody, 0)

    plsc.subcore_barrier()
    pltpu.sync_copy(msg_sh.at[pl.ds(nb, 640)],
                    msgp_hbm.at[c, pl.ds(nb, 640)])


_msg_kernel = pl.kernel(
    _msg_body,
    compiler_params=pltpu.CompilerParams(needs_layout_passes=False),
    out_type=jax.ShapeDtypeStruct((2, _NPAD, _H), jnp.float32),
    mesh=_mesh,
    scratch_types=[
        pltpu.VMEM((_SLAB,), jnp.int32),
        pltpu.VMEM((_NCH, _CHUNK), jnp.int32),
        pltpu.VMEM((_SLAB,), jnp.float32),
        pltpu.VMEM((_CHUNK, _H), jnp.float32),
        pltpu.VMEM((_CHUNK, _H), jnp.float32),
        pltpu.VMEM((_CHUNK, _H), jnp.float32),
        pltpu.VMEM((_CHUNK, _H), jnp.float32),
        pltpu.SemaphoreType.DMA,
        pltpu.SemaphoreType.DMA,
        pltpu.VMEM_SHARED((_NPAD, _H), jnp.float32),
    ],
)


def _k2_body(dp0, dp1, x, w, dis_out, gs_out):
    deg = dp0[...] + dp1[...] + 1.0
    dis = lax.rsqrt(jnp.maximum(deg, 1e-12))
    dis_out[...] = dis
    gs_out[...] = dis * jnp.dot(x[...], w[...],
                                preferred_element_type=jnp.float32)


_k2 = pl.pallas_call(
    _k2_body,
    grid=(_N // _BLK,),
    in_specs=[
        pl.BlockSpec((_BLK, 1), lambda i: (i, 0)),
        pl.BlockSpec((_BLK, 1), lambda i: (i, 0)),
        pl.BlockSpec((_BLK, _D), lambda i: (i, 0)),
        pl.BlockSpec((_D, _H), lambda i: (0, 0)),
    ],
    out_specs=[pl.BlockSpec((_BLK, 1), lambda i: (i, 0)),
               pl.BlockSpec((_BLK, _H), lambda i: (i, 0))],
    out_shape=[jax.ShapeDtypeStruct((_N, 1), jnp.float32),
               jax.ShapeDtypeStruct((_N, _H), jnp.float32)],
)


def _k4_body(dis, m0, m1, gs, b, w, gs2_out):
    h = jnp.maximum(dis[...] * (m0[...] + m1[...] + gs[...]) + b[...], 0.0)
    gs2_out[...] = dis[...] * jnp.dot(h, w[...],
                                      preferred_element_type=jnp.float32)


_k4 = pl.pallas_call(
    _k4_body,
    grid=(_N // _BLK,),
    in_specs=[
        pl.BlockSpec((_BLK, 1), lambda i: (i, 0)),
        pl.BlockSpec((_BLK, _H), lambda i: (i, 0)),
        pl.BlockSpec((_BLK, _H), lambda i: (i, 0)),
        pl.BlockSpec((_BLK, _H), lambda i: (i, 0)),
        pl.BlockSpec((1, _H), lambda i: (0, 0)),
        pl.BlockSpec((_H, _H), lambda i: (0, 0)),
    ],
    out_specs=pl.BlockSpec((_BLK, _H), lambda i: (i, 0)),
    out_shape=jax.ShapeDtypeStruct((_N, _H), jnp.float32),
)


def _k6_body(dis, m0, m1, gs, b, batch, lw, lb, out, sums, cnts):
    i = pl.program_id(0)

    @pl.when(i == 0)
    def _():
        sums[...] = jnp.zeros_like(sums)
        cnts[...] = jnp.zeros_like(cnts)

    h = jnp.maximum(dis[...] * (m0[...] + m1[...] + gs[...]) + b[...], 0.0)
    gi = lax.broadcasted_iota(jnp.int32, (_BLK, _G), 1)
    oh = (gi == batch[...]).astype(jnp.float32)
    dn = (((0,), (0,)), ((), ()))
    sums[...] += lax.dot_general(oh, h, dn,
                                 preferred_element_type=jnp.float32)
    cnts[...] += lax.dot_general(oh, jnp.ones((_BLK, 1), jnp.float32), dn,
                                 preferred_element_type=jnp.float32)

    @pl.when(i == pl.num_programs(0) - 1)
    def _():
        pooled = sums[...] / jnp.maximum(cnts[...], 1.0)
        out[...] = jnp.dot(pooled, lw[...],
                           preferred_element_type=jnp.float32) + lb[...]


_k6 = pl.pallas_call(
    _k6_body,
    grid=(_N // _BLK,),
    in_specs=[
        pl.BlockSpec((_BLK, 1), lambda i: (i, 0)),
        pl.BlockSpec((_BLK, _H), lambda i: (i, 0)),
        pl.BlockSpec((_BLK, _H), lambda i: (i, 0)),
        pl.BlockSpec((_BLK, _H), lambda i: (i, 0)),
        pl.BlockSpec((1, _H), lambda i: (0, 0)),
        pl.BlockSpec((_BLK, 1), lambda i: (i, 0)),
        pl.BlockSpec((_H, _C), lambda i: (0, 0)),
        pl.BlockSpec((1, _C), lambda i: (0, 0)),
    ],
    out_specs=pl.BlockSpec((_G, _C), lambda i: (0, 0)),
    out_shape=jax.ShapeDtypeStruct((_G, _C), jnp.float32),
    scratch_shapes=[pltpu.VMEM((_G, _H), jnp.float32),
                    pltpu.VMEM((_G, 1), jnp.float32)],
)


def kernel(x, edge_index, edge_attr, batch, ee_w1, ee_b1, ee_w2, ee_b2,
           conv1_w, conv1_b, stem_w, stem_b, lin_w, lin_b):
    npad = _EP - _E
    srcp = jnp.concatenate([edge_index[0], jnp.zeros((npad,), jnp.int32)])
    dstp = jnp.concatenate([edge_index[1],
                            jnp.full((npad,), _N + 200, jnp.int32)])
    attrp = jnp.concatenate([edge_attr[:, 0],
                             jnp.zeros((npad,), jnp.float32)])
    dst3d = dstp.reshape(32 * _NSLAB, _NCH, _CHUNK)
    par = jnp.concatenate([ee_w1[:, 0], ee_b1, ee_w2[0], ee_b2,
                           jnp.zeros((10,), jnp.float32)])

    ew1d, degp = _edge_kernel(srcp, dstp, attrp, par)
    dp0 = degp[0, 0, :_N, None]
    dp1 = degp[1, 0, :_N, None]
    dis, gs1 = _k2(dp0, dp1, x, conv1_w)

    zeros = jnp.zeros((_NPAD, _H), jnp.float32)
    msgp1 = _msg_kernel(srcp, dst3d, ew1d, gs1, zeros)
    gs2 = _k4(dis, msgp1[0, :_N], msgp1[1, :_N], gs1,
              conv1_b.reshape(1, _H), stem_w)
    msgp2 = _msg_kernel(srcp, dst3d, ew1d, gs2, zeros)
    out = _k6(dis, msgp2[0, :_N], msgp2[1, :_N], gs2, stem_b.reshape(1, _H),
              batch.reshape(_N, 1), lin_w, lin_b.reshape(1, _C))
    return out
